# Initial kernel scaffold; baseline (speedup 1.0000x reference)
import jax
import jax.numpy as jnp
from jax.experimental import pallas as pl


def kernel(hidden_states, expert_gate_weights):
    raise NotImplementedError("write your pallas kernel here")


# speedup vs baseline: 1.1454x; 1.1454x over previous
import functools

import jax
import jax.numpy as jnp
from jax.experimental import pallas as pl

TOP_K = 8
HIDDEN_SIZE = 1024
NUM_EXPERTS = 64


def _router_block(x_ref, w_ref, idx_ref, gates_ref):
    x = x_ref[...]
    w = w_ref[...]
    logits = jnp.dot(x, w, preferred_element_type=jnp.float32)
    m = jnp.max(logits, axis=-1, keepdims=True)
    e = jnp.exp(logits - m)
    denom = jnp.sum(e, axis=-1, keepdims=True)
    scores = e / denom

    expert_iota = jax.lax.broadcasted_iota(jnp.int32, scores.shape, 1)
    vals = []
    idxs = []
    s = scores
    for _ in range(TOP_K):
        v = jnp.max(s, axis=-1, keepdims=True)
        i = jnp.min(jnp.where(s == v, expert_iota, NUM_EXPERTS),
                    axis=-1, keepdims=True)
        vals.append(v)
        idxs.append(i)
        s = jnp.where(expert_iota == i, -jnp.inf, s)
    topv = jnp.concatenate(vals, axis=-1)
    topi = jnp.concatenate(idxs, axis=-1)
    gsum = jnp.sum(topv, axis=-1, keepdims=True)
    gates_ref[...] = topv / (gsum + 1e-06)
    idx_ref[...] = topi


@functools.partial(jax.jit, static_argnames=("block",))
def _router(flat_tokens, w_t, block=1024):
    n_tok = flat_tokens.shape[0]
    grid = (n_tok // block,)
    return pl.pallas_call(
        _router_block,
        grid=grid,
        in_specs=[
            pl.BlockSpec((block, HIDDEN_SIZE), lambda i: (i, 0)),
            pl.BlockSpec((HIDDEN_SIZE, NUM_EXPERTS), lambda i: (0, 0)),
        ],
        out_specs=[
            pl.BlockSpec((block, TOP_K), lambda i: (i, 0)),
            pl.BlockSpec((block, TOP_K), lambda i: (i, 0)),
        ],
        out_shape=[
            jax.ShapeDtypeStruct((n_tok, TOP_K), jnp.int32),
            jax.ShapeDtypeStruct((n_tok, TOP_K), jnp.float32),
        ],
    )(flat_tokens, w_t)


def kernel(hidden_states, expert_gate_weights):
    flat_tokens = hidden_states.reshape(-1, HIDDEN_SIZE)
    w_t = expert_gate_weights.T
    local_indices, topk_gates = _router(flat_tokens, w_t)
    return (local_indices, topk_gates)
